# baseline (speedup 1.0000x reference)
import jax
import jax.numpy as jnp
from jax import lax
from jax.experimental import pallas as pl
from jax.experimental.pallas import tpu as pltpu
from jax.experimental.pallas import tpu_sc as plsc

_B, _S, _L = 16, 2, 4096
_LOUT = 2064
_NC, _NS = 2, 16


def _body(x_hbm, tok_hbm, typ_hbm, buf_v, out_v):
    c = lax.axis_index("c")
    s = lax.axis_index("s")
    wid = s * _NC + c

    @pl.when(wid < _B)
    def _():
        pltpu.sync_copy(x_hbm.at[wid], buf_v)

        def fill(j, carry):
            out_v[pl.ds(j * jnp.int32(16), 16)] = buf_v[pl.ds(j * jnp.int32(16), 16)]
            return carry

        lax.fori_loop(jnp.int32(0), jnp.int32(2 * _LOUT // 16), fill, jnp.int32(0))
        pltpu.sync_copy(out_v, tok_hbm.at[wid])
        pltpu.sync_copy(out_v, typ_hbm.at[wid])


def kernel(inputs):
    xp = lax.bitcast_convert_type(inputs, jnp.int32).reshape(_B, 2 * _S * _L)
    mesh = plsc.VectorSubcoreMesh(
        core_axis_name="c", subcore_axis_name="s", num_cores=_NC, num_subcores=_NS
    )
    f = pl.kernel(
        _body,
        out_type=(
            jax.ShapeDtypeStruct((_B, 2 * _LOUT), jnp.int32),
            jax.ShapeDtypeStruct((_B, 2 * _LOUT), jnp.int32),
        ),
        mesh=mesh,
        compiler_params=pltpu.CompilerParams(needs_layout_passes=False),
        scratch_types=[pltpu.VMEM((2 * _S * _L,), jnp.int32),
                       pltpu.VMEM((2 * _LOUT,), jnp.int32)],
    )
    tok, typ = f(xp)
    tok64 = lax.bitcast_convert_type(tok.reshape(_B, _LOUT, 2), jnp.int64)
    typ64 = lax.bitcast_convert_type(typ.reshape(_B, _LOUT, 2), jnp.int64)
    return tok64[:, :2051], typ64[:, :2051]


# speedup vs baseline: 3.2154x; 3.2154x over previous
import jax
import jax.numpy as jnp
from jax import lax
from jax.experimental import pallas as pl
from jax.experimental.pallas import tpu as pltpu
from jax.experimental.pallas import tpu_sc as plsc

_BOS = 101
_EOS = 102
_M = 2048
_B, _S, _L = 16, 2, 4096
_LANES = 16
_NC, _NS = 2, 16
_LOUT = 2064
_NCHUNK = _L // _LANES
_CNT_UNROLL = 8
_SCAT_UNROLL = 4


def _preproc_body(x_hbm, out_hbm, row_v, out_v):
    c = lax.axis_index("c")
    s = lax.axis_index("s")
    wid = s * _NC + c

    @pl.when(wid < _B)
    def _():
        b = wid
        pltpu.sync_copy(x_hbm.at[b], row_v)

        lane = lax.iota(jnp.int32, _LANES)
        zero_v = jnp.zeros((_LANES,), jnp.int32)

        def count_seg(base):
            def cond(carry):
                j, cnt = carry
                return (j < _NCHUNK) & (cnt < _M)

            def body(carry):
                j, cnt = carry
                acc = zero_v
                for k in range(_CNT_UNROLL):
                    v = row_v[pl.ds(base + (j + jnp.int32(k)) * jnp.int32(_LANES), _LANES)]
                    acc = acc + plsc.all_reduce_population_count(v != 0)
                return j + jnp.int32(_CNT_UNROLL), cnt + jnp.max(acc)

            _, cnt = lax.while_loop(cond, body, (jnp.int32(0), jnp.int32(0)))
            return cnt

        l0 = count_seg(jnp.int32(0))
        l1 = count_seg(jnp.int32(_L))

        t0 = jnp.minimum(l0, jnp.where(2 * l1 >= _M, (_M + 1) // 2, _M - l1))
        t1 = jnp.minimum(l1, jnp.where(2 * l0 >= _M, _M // 2, _M - l0))

        def typ_fill(j, carry):
            p = j * jnp.int32(_LANES) + lane
            typ = jnp.where((p > t0 + 1) & (p <= t0 + t1 + 2),
                            jnp.int32(1), jnp.int32(0))
            out_v[pl.ds(jnp.int32(_LOUT) + j * jnp.int32(_LANES), _LANES)] = typ
            return carry

        lax.fori_loop(jnp.int32(0), jnp.int32(_LOUT // _LANES), typ_fill,
                      jnp.int32(0))

        def zero_fill(j, carry):
            out_v[pl.ds(j * jnp.int32(_LANES), _LANES)] = zero_v
            return carry

        lax.fori_loop((t0 + t1 + 3) // jnp.int32(_LANES),
                      jnp.int32(_LOUT // _LANES), zero_fill, jnp.int32(0))

        def scatter_seg(base, out_base, limit):
            def cond(carry):
                j, off = carry
                return (j < _NCHUNK) & (jnp.max(off) < limit)

            def body(carry):
                j, off = carry
                for k in range(_SCAT_UNROLL):
                    v = row_v[pl.ds(base + (j + jnp.int32(k)) * jnp.int32(_LANES), _LANES)]
                    m = v != 0
                    pos = plsc.cumsum(m.astype(jnp.int32)) + off
                    plsc.store_scatter(out_v, [out_base + pos - 1], v,
                                       mask=m & (pos <= limit))
                    off = off + plsc.all_reduce_population_count(m)
                return j + jnp.int32(_SCAT_UNROLL), off

            lax.while_loop(cond, body, (jnp.int32(0), zero_v))

        scatter_seg(jnp.int32(0), jnp.int32(1), t0)
        scatter_seg(jnp.int32(_L), t0 + 2, t1)

        sep_idx = jnp.where(lane == 0, jnp.int32(0),
                  jnp.where(lane == 1, t0 + 1, t0 + t1 + 2))
        sep_val = jnp.where(lane == 0, jnp.int32(_BOS), jnp.int32(_EOS))
        plsc.store_scatter(out_v, [sep_idx], sep_val, mask=lane < 3)

        pltpu.sync_copy(out_v, out_hbm.at[b])


def kernel(inputs):
    x32 = inputs.astype(jnp.int32).reshape(_B, _S * _L)
    mesh = plsc.VectorSubcoreMesh(
        core_axis_name="c", subcore_axis_name="s", num_cores=_NC, num_subcores=_NS
    )
    f = pl.kernel(
        _preproc_body,
        out_type=jax.ShapeDtypeStruct((_B, 2 * _LOUT), jnp.int32),
        mesh=mesh,
        compiler_params=pltpu.CompilerParams(needs_layout_passes=False),
        scratch_types=[
            pltpu.VMEM((_S * _L,), jnp.int32),
            pltpu.VMEM((2 * _LOUT,), jnp.int32),
        ],
    )
    out = f(x32)
    odt = inputs.dtype
    ncol = _M + 3
    return (out[:, :ncol].astype(odt),
            out[:, _LOUT:_LOUT + ncol].astype(odt))


# speedup vs baseline: 3.2368x; 1.0067x over previous
import jax
import jax.numpy as jnp
from jax import lax
from jax.experimental import pallas as pl
from jax.experimental.pallas import tpu as pltpu
from jax.experimental.pallas import tpu_sc as plsc

_BOS = 101
_EOS = 102
_M = 2048
_B, _S, _L = 16, 2, 4096
_LANES = 16
_NC, _NS = 2, 16
_LOUT = 2064
_NCHUNK = _L // _LANES
_CNT_UNROLL = 8
_SCAT_UNROLL = 4


def _preproc_body(x_hbm, out_hbm, row_v, out_v):
    c = lax.axis_index("c")
    s = lax.axis_index("s")
    wid = s * _NC + c

    @pl.when(wid < _B)
    def _():
        b = wid
        pltpu.sync_copy(x_hbm.at[b], row_v)

        lane = lax.iota(jnp.int32, _LANES)
        zero_v = jnp.zeros((_LANES,), jnp.int32)

        def count_seg(base):
            def cond(carry):
                j, cnt = carry
                return (j < _NCHUNK) & (cnt < _M)

            def body(carry):
                j, cnt = carry
                acc = zero_v
                for k in range(_CNT_UNROLL):
                    v = row_v[pl.ds(base + (j + jnp.int32(k)) * jnp.int32(_LANES), _LANES)]
                    acc = acc + plsc.all_reduce_population_count(v != 0)
                return j + jnp.int32(_CNT_UNROLL), cnt + jnp.max(acc)

            _, cnt = lax.while_loop(cond, body, (jnp.int32(0), jnp.int32(0)))
            return cnt

        l0 = count_seg(jnp.int32(0))
        l1 = count_seg(jnp.int32(_L))

        t0 = jnp.minimum(l0, jnp.where(2 * l1 >= _M, (_M + 1) // 2, _M - l1))
        t1 = jnp.minimum(l1, jnp.where(2 * l0 >= _M, _M // 2, _M - l0))

        def typ_fill(j, carry):
            p = j * jnp.int32(_LANES) + lane
            typ = jnp.where((p > t0 + 1) & (p <= t0 + t1 + 2),
                            jnp.int32(1), jnp.int32(0))
            out_v[pl.ds(jnp.int32(_LOUT) + j * jnp.int32(_LANES), _LANES)] = typ
            return carry

        lax.fori_loop(jnp.int32(0), jnp.int32(_LOUT // _LANES), typ_fill,
                      jnp.int32(0))

        def zero_fill(j, carry):
            out_v[pl.ds(j * jnp.int32(_LANES), _LANES)] = zero_v
            return carry

        lax.fori_loop((t0 + t1 + 3) // jnp.int32(_LANES),
                      jnp.int32(_LOUT // _LANES), zero_fill, jnp.int32(0))

        def scatter_seg(base, out_base, limit):
            def cond(carry):
                j, off = carry
                return (j < _NCHUNK) & (jnp.max(off) < limit)

            def body(carry):
                j, off = carry
                for k in range(_SCAT_UNROLL):
                    v = row_v[pl.ds(base + (j + jnp.int32(k)) * jnp.int32(_LANES), _LANES)]
                    m = v != 0
                    pos = plsc.cumsum(m.astype(jnp.int32)) + off
                    plsc.store_scatter(out_v, [out_base + pos - 1], v,
                                       mask=m & (pos <= limit))
                    off = off + plsc.all_reduce_population_count(m)
                return j + jnp.int32(_SCAT_UNROLL), off

            lax.while_loop(cond, body, (jnp.int32(0), zero_v))

        scatter_seg(jnp.int32(0), jnp.int32(1), t0)
        scatter_seg(jnp.int32(_L), t0 + 2, t1)

        sep_idx = jnp.where(lane == 0, jnp.int32(0),
                  jnp.where(lane == 1, t0 + 1, t0 + t1 + 2))
        sep_val = jnp.where(lane == 0, jnp.int32(_BOS), jnp.int32(_EOS))
        plsc.store_scatter(out_v, [sep_idx], sep_val, mask=lane < 3)

        pltpu.sync_copy(out_v, out_hbm.at[b])


def kernel(inputs):
    x2d = inputs.astype(jnp.int32).reshape(_B, _S * _L)
    mesh = plsc.VectorSubcoreMesh(
        core_axis_name="c", subcore_axis_name="s", num_cores=_NC, num_subcores=_NS
    )
    f = pl.kernel(
        _preproc_body,
        out_type=jax.ShapeDtypeStruct((_B, 2 * _LOUT), jnp.int32),
        mesh=mesh,
        compiler_params=pltpu.CompilerParams(needs_layout_passes=False),
        scratch_types=[
            pltpu.VMEM((_S * _L,), jnp.int32),
            pltpu.VMEM((2 * _LOUT,), jnp.int32),
        ],
    )
    out = f(x2d)
    odt = inputs.dtype
    ncol = _M + 3
    return (out[:, :ncol].astype(odt),
            out[:, _LOUT:_LOUT + ncol].astype(odt))
